# baseline (device time: 108045 ns/iter reference)
import jax
import jax.numpy as jnp
from jax import lax
from jax.experimental import pallas as pl
from jax.experimental.pallas import tpu as pltpu

N_DEV = 8
B = 2
SQ = 256
DM = 512
DH = 64
HQ_LOC = 4
HQ = N_DEV * HQ_LOC
SKV_LOC = 256
BLK = 64
BF16 = jnp.bfloat16


def kernel(x, Wq, K_ext, V_ext, Wo):
    def body(x_ref, wq_ref, k_ref, v_ref, wo_ref, out_ref,
             kstage, vstage, kbuf, vbuf, part, psum,
             ksend_sems, vsend_sems, krecv_sem, vrecv_sem,
             psend_sems, precv_sems):
        me = lax.axis_index("i")

        @pl.when(me == 0)
        def _():
            kstage[...] = k_ref[...].astype(BF16)
            vstage[...] = v_ref[...].astype(BF16)
            for d in range(1, N_DEV):
                pltpu.make_async_remote_copy(
                    src_ref=kstage.at[:, :, pl.ds(HQ_LOC * d, HQ_LOC), :],
                    dst_ref=kbuf,
                    send_sem=ksend_sems.at[d],
                    recv_sem=krecv_sem,
                    device_id=(d,),
                    device_id_type=pl.DeviceIdType.MESH,
                ).start()
                pltpu.make_async_remote_copy(
                    src_ref=vstage.at[:, :, pl.ds(HQ_LOC * d, HQ_LOC), :],
                    dst_ref=vbuf,
                    send_sem=vsend_sems.at[d],
                    recv_sem=vrecv_sem,
                    device_id=(d,),
                    device_id_type=pl.DeviceIdType.MESH,
                ).start()
            kbuf[...] = kstage[:, :, 0:HQ_LOC, :]
            vbuf[...] = vstage[:, :, 0:HQ_LOC, :]

        wq = wq_ref[...].astype(BF16)
        qs = []
        for b in range(B):
            xb = x_ref[b].astype(BF16)
            q = jnp.dot(xb, wq, preferred_element_type=jnp.float32)
            qs.append((q * 0.125).astype(BF16))

        @pl.when(me != 0)
        def _():
            pltpu.make_async_remote_copy(
                src_ref=kbuf, dst_ref=kbuf,
                send_sem=ksend_sems.at[0], recv_sem=krecv_sem,
                device_id=(0,), device_id_type=pl.DeviceIdType.MESH,
            ).wait_recv()
            pltpu.make_async_remote_copy(
                src_ref=vbuf, dst_ref=vbuf,
                send_sem=vsend_sems.at[0], recv_sem=vrecv_sem,
                device_id=(0,), device_id_type=pl.DeviceIdType.MESH,
            ).wait_recv()

        @pl.when(me == 0)
        def _():
            for d in range(1, N_DEV):
                pltpu.make_async_remote_copy(
                    src_ref=kstage.at[:, :, pl.ds(HQ_LOC * d, HQ_LOC), :],
                    dst_ref=kbuf,
                    send_sem=ksend_sems.at[d],
                    recv_sem=krecv_sem,
                    device_id=(d,),
                    device_id_type=pl.DeviceIdType.MESH,
                ).wait_send()
                pltpu.make_async_remote_copy(
                    src_ref=vstage.at[:, :, pl.ds(HQ_LOC * d, HQ_LOC), :],
                    dst_ref=vbuf,
                    send_sem=vsend_sems.at[d],
                    recv_sem=vrecv_sem,
                    device_id=(d,),
                    device_id_type=pl.DeviceIdType.MESH,
                ).wait_send()

        row = lax.broadcasted_iota(jnp.int32, (SQ, SKV_LOC), 0) // BLK
        col = lax.broadcasted_iota(jnp.int32, (SQ, SKV_LOC), 1) // BLK
        mask = col <= row
        wo = wo_ref[...].astype(BF16)
        for b in range(B):
            acc = jnp.zeros((SQ, DM), jnp.float32)
            for h in range(HQ_LOC):
                qh = qs[b][:, DH * h:DH * (h + 1)]
                kh = kbuf[b, :, h, :]
                s = lax.dot_general(
                    qh, kh, (((1,), (1,)), ((), ())),
                    preferred_element_type=jnp.float32)
                s = jnp.where(mask, s, -1e9)
                mx = jnp.max(s, axis=1, keepdims=True)
                w = jnp.exp(s - mx)
                w = (w / jnp.sum(w, axis=1, keepdims=True)).astype(BF16)
                ch = jnp.dot(w, vbuf[b, :, h, :],
                             preferred_element_type=jnp.float32)
                acc = acc + jnp.dot(
                    ch.astype(BF16), wo[DH * h:DH * (h + 1), :],
                    preferred_element_type=jnp.float32)
            part[b] = acc.astype(BF16)

        for s in range(N_DEV):
            @pl.when(me == s)
            def _(s=s):
                psum[s] = part[...]
                for d in range(N_DEV):
                    if d == s:
                        continue
                    pltpu.make_async_remote_copy(
                        src_ref=part, dst_ref=psum.at[s],
                        send_sem=psend_sems.at[d], recv_sem=precv_sems.at[s],
                        device_id=(d,), device_id_type=pl.DeviceIdType.MESH,
                    ).start()

        for s in range(N_DEV):
            @pl.when(me != s)
            def _(s=s):
                pltpu.make_async_remote_copy(
                    src_ref=part, dst_ref=psum.at[s],
                    send_sem=psend_sems.at[0], recv_sem=precv_sems.at[s],
                    device_id=(0,), device_id_type=pl.DeviceIdType.MESH,
                ).wait_recv()

        for s in range(N_DEV):
            @pl.when(me == s)
            def _(s=s):
                for d in range(N_DEV):
                    if d == s:
                        continue
                    pltpu.make_async_remote_copy(
                        src_ref=part, dst_ref=psum.at[s],
                        send_sem=psend_sems.at[d], recv_sem=precv_sems.at[s],
                        device_id=(d,), device_id_type=pl.DeviceIdType.MESH,
                    ).wait_send()

        total = psum[0].astype(jnp.float32)
        for s in range(1, N_DEV):
            total = total + psum[s].astype(jnp.float32)
        out_ref[...] = total

    return pl.pallas_call(
        body,
        out_shape=jax.ShapeDtypeStruct((B, SQ, DM), jnp.float32),
        in_specs=[pl.BlockSpec(memory_space=pltpu.VMEM)] * 5,
        out_specs=pl.BlockSpec(memory_space=pltpu.VMEM),
        scratch_shapes=[
            pltpu.VMEM((B, SKV_LOC, HQ, DH), BF16),
            pltpu.VMEM((B, SKV_LOC, HQ, DH), BF16),
            pltpu.VMEM((B, SKV_LOC, HQ_LOC, DH), BF16),
            pltpu.VMEM((B, SKV_LOC, HQ_LOC, DH), BF16),
            pltpu.VMEM((B, SQ, DM), BF16),
            pltpu.VMEM((N_DEV, B, SQ, DM), BF16),
            pltpu.SemaphoreType.DMA((N_DEV,)),
            pltpu.SemaphoreType.DMA((N_DEV,)),
            pltpu.SemaphoreType.DMA,
            pltpu.SemaphoreType.DMA,
            pltpu.SemaphoreType.DMA((N_DEV,)),
            pltpu.SemaphoreType.DMA((N_DEV,)),
        ],
    )(x, Wq, K_ext, V_ext, Wo)


# device time: 87352 ns/iter; 1.2369x vs baseline; 1.2369x over previous
import jax
import jax.numpy as jnp
from jax import lax
from jax.experimental import pallas as pl
from jax.experimental.pallas import tpu as pltpu

N_DEV = 8
B = 2
SQ = 256
DM = 512
DH = 64
HQ_LOC = 4
HQ = N_DEV * HQ_LOC
SKV_LOC = 256
BLK = 64
BF16 = jnp.bfloat16


def kernel(x, Wq, K_ext, V_ext, Wo):
    SC = SQ // N_DEV

    BCAST_ORDER = (6, 2, 5, 7, 1, 3, 4)

    def body(x_ref, wq_ref, k_ref, v_ref, wo_ref, out_ref,
             kstage, vstage, kbuf, vbuf, part, rsbuf, agstage, agbuf,
             ksend_sems, vsend_sems, krecv_sem, vrecv_sem,
             rssend_sems, rsrecv_sems, agsend_sems, agrecv_sems):
        me = lax.axis_index("i")

        @pl.when(me == 0)
        def _():
            kstage[...] = k_ref[...].astype(BF16)
            vstage[...] = v_ref[...].astype(BF16)
            for d in BCAST_ORDER:
                pltpu.make_async_remote_copy(
                    src_ref=kstage.at[:, :, pl.ds(HQ_LOC * d, HQ_LOC), :],
                    dst_ref=kbuf,
                    send_sem=ksend_sems.at[d],
                    recv_sem=krecv_sem,
                    device_id=(d,),
                    device_id_type=pl.DeviceIdType.MESH,
                ).start()
                pltpu.make_async_remote_copy(
                    src_ref=vstage.at[:, :, pl.ds(HQ_LOC * d, HQ_LOC), :],
                    dst_ref=vbuf,
                    send_sem=vsend_sems.at[d],
                    recv_sem=vrecv_sem,
                    device_id=(d,),
                    device_id_type=pl.DeviceIdType.MESH,
                ).start()
            kbuf[...] = kstage[:, :, 0:HQ_LOC, :]
            vbuf[...] = vstage[:, :, 0:HQ_LOC, :]

        wq = wq_ref[...].astype(BF16)
        qs = []
        for b in range(B):
            xb = x_ref[b].astype(BF16)
            q = jnp.dot(xb, wq, preferred_element_type=jnp.float32)
            qs.append((q * 0.125).astype(BF16))

        @pl.when(me != 0)
        def _():
            pltpu.make_async_remote_copy(
                src_ref=kbuf, dst_ref=kbuf,
                send_sem=ksend_sems.at[0], recv_sem=krecv_sem,
                device_id=(0,), device_id_type=pl.DeviceIdType.MESH,
            ).wait_recv()
            pltpu.make_async_remote_copy(
                src_ref=vbuf, dst_ref=vbuf,
                send_sem=vsend_sems.at[0], recv_sem=vrecv_sem,
                device_id=(0,), device_id_type=pl.DeviceIdType.MESH,
            ).wait_recv()

        @pl.when(me == 0)
        def _():
            for d in range(1, N_DEV):
                pltpu.make_async_remote_copy(
                    src_ref=kstage.at[:, :, pl.ds(HQ_LOC * d, HQ_LOC), :],
                    dst_ref=kbuf,
                    send_sem=ksend_sems.at[d],
                    recv_sem=krecv_sem,
                    device_id=(d,),
                    device_id_type=pl.DeviceIdType.MESH,
                ).wait_send()
                pltpu.make_async_remote_copy(
                    src_ref=vstage.at[:, :, pl.ds(HQ_LOC * d, HQ_LOC), :],
                    dst_ref=vbuf,
                    send_sem=vsend_sems.at[d],
                    recv_sem=vrecv_sem,
                    device_id=(d,),
                    device_id_type=pl.DeviceIdType.MESH,
                ).wait_send()

        row = lax.broadcasted_iota(jnp.int32, (SQ, SKV_LOC), 0) // BLK
        col = lax.broadcasted_iota(jnp.int32, (SQ, SKV_LOC), 1) // BLK
        mask = col <= row
        wo = wo_ref[...].astype(BF16)
        for b in range(B):
            acc = jnp.zeros((SQ, DM), jnp.float32)
            for h in range(HQ_LOC):
                qh = qs[b][:, DH * h:DH * (h + 1)]
                kh = kbuf[b, :, h, :]
                s = lax.dot_general(
                    qh, kh, (((1,), (1,)), ((), ())),
                    preferred_element_type=jnp.float32)
                s = jnp.where(mask, s, -1e9)
                mx = jnp.max(s, axis=1, keepdims=True)
                w = jnp.exp(s - mx)
                w = (w / jnp.sum(w, axis=1, keepdims=True)).astype(BF16)
                ch = jnp.dot(w, vbuf[b, :, h, :],
                             preferred_element_type=jnp.float32)
                acc = acc + jnp.dot(
                    ch.astype(BF16), wo[DH * h:DH * (h + 1), :],
                    preferred_element_type=jnp.float32)
            part[b] = acc.astype(BF16)

        for s in range(N_DEV):
            @pl.when(me == s)
            def _(s=s):
                rsbuf[s] = part[:, SC * s:SC * (s + 1), :]
                for d in range(N_DEV):
                    if d == s:
                        continue
                    pltpu.make_async_remote_copy(
                        src_ref=part.at[:, pl.ds(SC * d, SC), :],
                        dst_ref=rsbuf.at[s],
                        send_sem=rssend_sems.at[d], recv_sem=rsrecv_sems.at[s],
                        device_id=(d,), device_id_type=pl.DeviceIdType.MESH,
                    ).start()

        for s in range(N_DEV):
            @pl.when(me != s)
            def _(s=s):
                pltpu.make_async_remote_copy(
                    src_ref=rsbuf.at[s], dst_ref=rsbuf.at[s],
                    send_sem=rssend_sems.at[0], recv_sem=rsrecv_sems.at[s],
                    device_id=(0,), device_id_type=pl.DeviceIdType.MESH,
                ).wait_recv()

        red = rsbuf[0].astype(jnp.float32)
        for s in range(1, N_DEV):
            red = red + rsbuf[s].astype(jnp.float32)
        agstage[...] = red.astype(BF16)

        for s in range(N_DEV):
            @pl.when(me == s)
            def _(s=s):
                out_ref[:, SC * s:SC * (s + 1), :] = red
                for d in range(N_DEV):
                    if d == s:
                        continue
                    pltpu.make_async_remote_copy(
                        src_ref=agstage, dst_ref=agbuf.at[s],
                        send_sem=agsend_sems.at[d], recv_sem=agrecv_sems.at[s],
                        device_id=(d,), device_id_type=pl.DeviceIdType.MESH,
                    ).start()

        for s in range(N_DEV):
            @pl.when(me == s)
            def _(s=s):
                for d in range(N_DEV):
                    if d == s:
                        continue
                    pltpu.make_async_remote_copy(
                        src_ref=part.at[:, pl.ds(SC * d, SC), :],
                        dst_ref=rsbuf.at[s],
                        send_sem=rssend_sems.at[d], recv_sem=rsrecv_sems.at[s],
                        device_id=(d,), device_id_type=pl.DeviceIdType.MESH,
                    ).wait_send()

        for s in range(N_DEV):
            @pl.when(me != s)
            def _(s=s):
                pltpu.make_async_remote_copy(
                    src_ref=agbuf.at[s], dst_ref=agbuf.at[s],
                    send_sem=agsend_sems.at[0], recv_sem=agrecv_sems.at[s],
                    device_id=(0,), device_id_type=pl.DeviceIdType.MESH,
                ).wait_recv()
                out_ref[:, SC * s:SC * (s + 1), :] = agbuf[s].astype(jnp.float32)

        for s in range(N_DEV):
            @pl.when(me == s)
            def _(s=s):
                for d in range(N_DEV):
                    if d == s:
                        continue
                    pltpu.make_async_remote_copy(
                        src_ref=agstage, dst_ref=agbuf.at[s],
                        send_sem=agsend_sems.at[d], recv_sem=agrecv_sems.at[s],
                        device_id=(d,), device_id_type=pl.DeviceIdType.MESH,
                    ).wait_send()

    return pl.pallas_call(
        body,
        out_shape=jax.ShapeDtypeStruct((B, SQ, DM), jnp.float32),
        in_specs=[pl.BlockSpec(memory_space=pltpu.VMEM)] * 5,
        out_specs=pl.BlockSpec(memory_space=pltpu.VMEM),
        scratch_shapes=[
            pltpu.VMEM((B, SKV_LOC, HQ, DH), BF16),
            pltpu.VMEM((B, SKV_LOC, HQ, DH), BF16),
            pltpu.VMEM((B, SKV_LOC, HQ_LOC, DH), BF16),
            pltpu.VMEM((B, SKV_LOC, HQ_LOC, DH), BF16),
            pltpu.VMEM((B, SQ, DM), BF16),
            pltpu.VMEM((N_DEV, B, SQ // N_DEV, DM), BF16),
            pltpu.VMEM((B, SQ // N_DEV, DM), BF16),
            pltpu.VMEM((N_DEV, B, SQ // N_DEV, DM), BF16),
            pltpu.SemaphoreType.DMA((N_DEV,)),
            pltpu.SemaphoreType.DMA((N_DEV,)),
            pltpu.SemaphoreType.DMA,
            pltpu.SemaphoreType.DMA,
            pltpu.SemaphoreType.DMA((N_DEV,)),
            pltpu.SemaphoreType.DMA((N_DEV,)),
            pltpu.SemaphoreType.DMA((N_DEV,)),
            pltpu.SemaphoreType.DMA((N_DEV,)),
        ],
    )(x, Wq, K_ext, V_ext, Wo)


# device time: 77006 ns/iter; 1.4031x vs baseline; 1.1344x over previous
import jax
import jax.numpy as jnp
from jax import lax
from jax.experimental import pallas as pl
from jax.experimental.pallas import tpu as pltpu

N_DEV = 8
B = 2
SQ = 256
DM = 512
DH = 64
HQ_LOC = 4
HQ = N_DEV * HQ_LOC
SKV_LOC = 256
BLK = 64
BF16 = jnp.bfloat16
SC = SQ // N_DEV

MESH = pl.DeviceIdType.MESH


def kernel(x, Wq, K_ext, V_ext, Wo):
    def body(x_ref, wq_ref, k_ref, v_ref, wo_ref, out_ref,
             kf32, vf32, kstage, vstage, kbuf, vbuf, rbufs,
             part, rsbuf, agstage, agbuf,
             copy_sems, ksend_sems, vsend_sems, krecv_sem, vrecv_sem,
             relay_recv_sems, fwd_send_sems,
             rssend_sems, rsrecv_sems, agsend_sems, agrecv_sems):
        me = lax.axis_index("i")

        def slice_rdma(stage, d, dst, send_sem, recv_sem, target):
            return pltpu.make_async_remote_copy(
                src_ref=stage.at[:, :, pl.ds(HQ_LOC * d, HQ_LOC), :],
                dst_ref=dst, send_sem=send_sem, recv_sem=recv_sem,
                device_id=(target,), device_id_type=MESH)

        def kdir(d):
            return slice_rdma(kstage, d, kbuf, ksend_sems.at[d], krecv_sem, d)

        def vdir(d):
            return slice_rdma(vstage, d, vbuf, vsend_sems.at[d], vrecv_sem, d)

        def fwd(slot, dst, recv_sem, target):
            return pltpu.make_async_remote_copy(
                src_ref=rbufs.at[slot], dst_ref=dst,
                send_sem=fwd_send_sems.at[slot], recv_sem=recv_sem,
                device_id=(target,), device_id_type=MESH)

        def relay_wait(slot):
            pltpu.make_async_remote_copy(
                src_ref=rbufs.at[slot], dst_ref=rbufs.at[slot],
                send_sem=fwd_send_sems.at[slot],
                recv_sem=relay_recv_sems.at[slot],
                device_id=(0,), device_id_type=MESH).wait_recv()

        def ph1_bcast_issue():
            @pl.when(me == 0)
            def _():
                ck = pltpu.make_async_copy(k_ref, kf32, copy_sems.at[0])
                cv = pltpu.make_async_copy(v_ref, vf32, copy_sems.at[1])
                ck.start()
                cv.start()
                ck.wait()
                cv.wait()
                kstage[...] = kf32[...].astype(BF16)
                vstage[...] = vf32[...].astype(BF16)

                slice_rdma(kstage, 5, rbufs.at[0], ksend_sems.at[5],
                           relay_recv_sems.at[0], 4).start()
                slice_rdma(vstage, 5, rbufs.at[1], vsend_sems.at[5],
                           relay_recv_sems.at[1], 4).start()
                slice_rdma(vstage, 7, rbufs.at[2], vsend_sems.at[7],
                           relay_recv_sems.at[2], 4).start()
                slice_rdma(kstage, 6, rbufs.at[0], ksend_sems.at[6],
                           relay_recv_sems.at[0], 3).start()
                slice_rdma(vstage, 6, rbufs.at[1], vsend_sems.at[6],
                           relay_recv_sems.at[1], 3).start()
                kdir(2).start()
                vdir(2).start()
                kdir(1).start()
                vdir(1).start()

                slice_rdma(kstage, 5, rbufs.at[0], ksend_sems.at[5],
                           relay_recv_sems.at[0], 4).wait_send()
                slice_rdma(vstage, 5, rbufs.at[1], vsend_sems.at[5],
                           relay_recv_sems.at[1], 4).wait_send()
                slice_rdma(vstage, 7, rbufs.at[2], vsend_sems.at[7],
                           relay_recv_sems.at[2], 4).wait_send()
                kdir(4).start()
                vdir(4).start()

                slice_rdma(kstage, 6, rbufs.at[0], ksend_sems.at[6],
                           relay_recv_sems.at[0], 3).wait_send()
                slice_rdma(vstage, 6, rbufs.at[1], vsend_sems.at[6],
                           relay_recv_sems.at[1], 3).wait_send()
                kdir(3).start()
                vdir(3).start()
                kdir(7).start()

                kbuf[...] = kstage[:, :, 0:HQ_LOC, :]
                vbuf[...] = vstage[:, :, 0:HQ_LOC, :]

        def ph1_relay():
            @pl.when(me == 4)
            def _():
                relay_wait(0)
                fwd(0, kbuf, krecv_sem, 5).start()
                relay_wait(1)
                fwd(1, vbuf, vrecv_sem, 5).start()
                relay_wait(2)
                fwd(2, vbuf, vrecv_sem, 7).start()

            @pl.when(me == 3)
            def _():
                relay_wait(0)
                fwd(0, kbuf, krecv_sem, 6).start()
                relay_wait(1)
                fwd(1, vbuf, vrecv_sem, 6).start()

        def ph2_qproj():
            wq = wq_ref[...].astype(BF16)
            qs = []
            for b in range(B):
                xb = x_ref[b].astype(BF16)
                q = jnp.dot(xb, wq, preferred_element_type=jnp.float32)
                qs.append((q * 0.125).astype(BF16))
            return qs

        def ph_kv_wait():
            @pl.when(me != 0)
            def _():
                pltpu.make_async_remote_copy(
                    src_ref=kbuf, dst_ref=kbuf,
                    send_sem=ksend_sems.at[0], recv_sem=krecv_sem,
                    device_id=(0,), device_id_type=MESH).wait_recv()
                pltpu.make_async_remote_copy(
                    src_ref=vbuf, dst_ref=vbuf,
                    send_sem=vsend_sems.at[0], recv_sem=vrecv_sem,
                    device_id=(0,), device_id_type=MESH).wait_recv()

        def ph3_attn(qs):
            row = lax.broadcasted_iota(jnp.int32, (SQ, SKV_LOC), 0) // BLK
            col = lax.broadcasted_iota(jnp.int32, (SQ, SKV_LOC), 1) // BLK
            mask = col <= row
            wo = wo_ref[...].astype(BF16)
            for b in range(B):
                acc = jnp.zeros((SQ, DM), jnp.float32)
                for h in range(HQ_LOC):
                    qh = qs[b][:, DH * h:DH * (h + 1)]
                    kh = kbuf[b, :, h, :]
                    s = lax.dot_general(
                        qh, kh, (((1,), (1,)), ((), ())),
                        preferred_element_type=jnp.float32)
                    s = jnp.where(mask, s, -1e9)
                    mx = jnp.max(s, axis=1, keepdims=True)
                    w = jnp.exp(s - mx)
                    w = (w / jnp.sum(w, axis=1, keepdims=True)).astype(BF16)
                    ch = jnp.dot(w, vbuf[b, :, h, :],
                                 preferred_element_type=jnp.float32)
                    acc = acc + jnp.dot(
                        ch.astype(BF16), wo[DH * h:DH * (h + 1), :],
                        preferred_element_type=jnp.float32)
                part[b] = acc.astype(BF16)

        def rs_rdma(s, d):
            return pltpu.make_async_remote_copy(
                src_ref=part.at[:, pl.ds(SC * d, SC), :],
                dst_ref=rsbuf.at[s],
                send_sem=rssend_sems.at[d], recv_sem=rsrecv_sems.at[s],
                device_id=(d,), device_id_type=MESH)

        def ag_rdma(s, d):
            return pltpu.make_async_remote_copy(
                src_ref=agstage, dst_ref=agbuf.at[s],
                send_sem=agsend_sems.at[d], recv_sem=agrecv_sems.at[s],
                device_id=(d,), device_id_type=MESH)

        def ph4_rs_issue():
            for s in range(N_DEV):
                @pl.when(me == s)
                def _(s=s):
                    rsbuf[s] = part[:, SC * s:SC * (s + 1), :]
                    for d in range(N_DEV):
                        if d != s:
                            rs_rdma(s, d).start()

        def ph4_rs_wait():
            for s in range(N_DEV):
                @pl.when(me != s)
                def _(s=s):
                    rs_rdma(s, 0).wait_recv()

        def ph4_reduce():
            red = rsbuf[0].astype(jnp.float32)
            for s in range(1, N_DEV):
                red = red + rsbuf[s].astype(jnp.float32)
            agstage[...] = red.astype(BF16)
            return red

        def ph4_ag_issue(red):
            for s in range(N_DEV):
                @pl.when(me == s)
                def _(s=s):
                    out_ref[:, SC * s:SC * (s + 1), :] = red
                    for d in range(N_DEV):
                        if d != s:
                            ag_rdma(s, d).start()
            for s in range(N_DEV):
                @pl.when(me == s)
                def _(s=s):
                    for d in range(N_DEV):
                        if d != s:
                            rs_rdma(s, d).wait_send()

        def ph4_ag_wait():
            for s in range(N_DEV):
                @pl.when(me != s)
                def _(s=s):
                    ag_rdma(s, 0).wait_recv()
                    out_ref[:, SC * s:SC * (s + 1), :] = (
                        agbuf[s].astype(jnp.float32))
            for s in range(N_DEV):
                @pl.when(me == s)
                def _(s=s):
                    for d in range(N_DEV):
                        if d != s:
                            ag_rdma(s, d).wait_send()

        def ph5_drain():
            @pl.when(me == 0)
            def _():
                for d in (1, 2, 3, 4):
                    kdir(d).wait_send()
                    vdir(d).wait_send()
                kdir(7).wait_send()

            @pl.when(me == 4)
            def _():
                fwd(0, kbuf, krecv_sem, 5).wait_send()
                fwd(1, vbuf, vrecv_sem, 5).wait_send()
                fwd(2, vbuf, vrecv_sem, 7).wait_send()

            @pl.when(me == 3)
            def _():
                fwd(0, kbuf, krecv_sem, 6).wait_send()
                fwd(1, vbuf, vrecv_sem, 6).wait_send()

        with jax.named_scope("p1_bcast_issue"):
            ph1_bcast_issue()
        with jax.named_scope("p1_relay"):
            ph1_relay()
        with jax.named_scope("p2_qproj"):
            qs = ph2_qproj()
        with jax.named_scope("p2_kv_wait"):
            ph_kv_wait()
        with jax.named_scope("p3_attn"):
            ph3_attn(qs)
        with jax.named_scope("p4_rs_issue"):
            ph4_rs_issue()
        with jax.named_scope("p4_rs_wait"):
            ph4_rs_wait()
        with jax.named_scope("p4_reduce"):
            red = ph4_reduce()
        with jax.named_scope("p4_ag_issue"):
            ph4_ag_issue(red)
        with jax.named_scope("p4_ag_wait"):
            ph4_ag_wait()
        with jax.named_scope("p5_drain"):
            ph5_drain()

    return pl.pallas_call(
        body,
        out_shape=jax.ShapeDtypeStruct((B, SQ, DM), jnp.float32),
        in_specs=[
            pl.BlockSpec(memory_space=pltpu.VMEM),
            pl.BlockSpec(memory_space=pltpu.VMEM),
            pl.BlockSpec(memory_space=pl.ANY),
            pl.BlockSpec(memory_space=pl.ANY),
            pl.BlockSpec(memory_space=pltpu.VMEM),
        ],
        out_specs=pl.BlockSpec(memory_space=pltpu.VMEM),
        scratch_shapes=[
            pltpu.VMEM((B, SKV_LOC, HQ, DH), jnp.float32),
            pltpu.VMEM((B, SKV_LOC, HQ, DH), jnp.float32),
            pltpu.VMEM((B, SKV_LOC, HQ, DH), BF16),
            pltpu.VMEM((B, SKV_LOC, HQ, DH), BF16),
            pltpu.VMEM((B, SKV_LOC, HQ_LOC, DH), BF16),
            pltpu.VMEM((B, SKV_LOC, HQ_LOC, DH), BF16),
            pltpu.VMEM((3, B, SKV_LOC, HQ_LOC, DH), BF16),
            pltpu.VMEM((B, SQ, DM), BF16),
            pltpu.VMEM((N_DEV, B, SC, DM), BF16),
            pltpu.VMEM((B, SC, DM), BF16),
            pltpu.VMEM((N_DEV, B, SC, DM), BF16),
            pltpu.SemaphoreType.DMA((2,)),
            pltpu.SemaphoreType.DMA((N_DEV,)),
            pltpu.SemaphoreType.DMA((N_DEV,)),
            pltpu.SemaphoreType.DMA,
            pltpu.SemaphoreType.DMA,
            pltpu.SemaphoreType.DMA((3,)),
            pltpu.SemaphoreType.DMA((3,)),
            pltpu.SemaphoreType.DMA((N_DEV,)),
            pltpu.SemaphoreType.DMA((N_DEV,)),
            pltpu.SemaphoreType.DMA((N_DEV,)),
            pltpu.SemaphoreType.DMA((N_DEV,)),
        ],
    )(x, Wq, K_ext, V_ext, Wo)


# device time: 46818 ns/iter; 2.3078x vs baseline; 1.6448x over previous
import jax
import jax.numpy as jnp
from jax import lax
from jax.experimental import pallas as pl
from jax.experimental.pallas import tpu as pltpu

N_DEV = 8
B = 2
SQ = 256
DM = 512
DH = 64
HQ_LOC = 4
HQ = N_DEV * HQ_LOC
SKV_LOC = 256
BLK = 64
BF16 = jnp.bfloat16
SC = SQ // N_DEV

MESH = pl.DeviceIdType.MESH


def kernel(x, Wq, K_ext, V_ext, Wo):
    def body(x_ref, wq_ref, k_ref, v_ref, wo_ref, out_ref,
             kf32, vf32, kstage, vstage, kbuf, vbuf, rbufs,
             part, rsbuf, agstage, agbuf,
             copy_sems, ksend_sems, vsend_sems, krecv_sem, vrecv_sem,
             relay_recv_sems, fwd_send_sems,
             rssend_sems, rsrecv_sems, agsend_sems, agrecv_sems):
        me = lax.axis_index("i")

        def slice_rdma(stage, d, dst, send_sem, recv_sem, target):
            return pltpu.make_async_remote_copy(
                src_ref=stage.at[:, :, pl.ds(HQ_LOC * DH * d, HQ_LOC * DH)],
                dst_ref=dst, send_sem=send_sem, recv_sem=recv_sem,
                device_id=(target,), device_id_type=MESH)

        def kdir(d):
            return slice_rdma(kstage, d, kbuf, ksend_sems.at[d], krecv_sem, d)

        def vdir(d):
            return slice_rdma(vstage, d, vbuf, vsend_sems.at[d], vrecv_sem, d)

        def fwd(slot, dst, recv_sem, target):
            return pltpu.make_async_remote_copy(
                src_ref=rbufs.at[slot], dst_ref=dst,
                send_sem=fwd_send_sems.at[slot], recv_sem=recv_sem,
                device_id=(target,), device_id_type=MESH)

        def relay_wait(slot):
            pltpu.make_async_remote_copy(
                src_ref=rbufs.at[slot], dst_ref=rbufs.at[slot],
                send_sem=fwd_send_sems.at[slot],
                recv_sem=relay_recv_sems.at[slot],
                device_id=(0,), device_id_type=MESH).wait_recv()

        def ph1_bcast_issue():
            @pl.when(me == 0)
            def _():
                ck = pltpu.make_async_copy(k_ref, kf32, copy_sems.at[0])
                cv = pltpu.make_async_copy(v_ref, vf32, copy_sems.at[1])
                ck.start()
                cv.start()
                ck.wait()
                cv.wait()
                kstage[...] = kf32[...].astype(BF16)
                vstage[...] = vf32[...].astype(BF16)

                slice_rdma(kstage, 5, rbufs.at[0], ksend_sems.at[5],
                           relay_recv_sems.at[0], 4).start()
                slice_rdma(vstage, 5, rbufs.at[1], vsend_sems.at[5],
                           relay_recv_sems.at[1], 4).start()
                slice_rdma(vstage, 7, rbufs.at[2], vsend_sems.at[7],
                           relay_recv_sems.at[2], 4).start()
                slice_rdma(kstage, 6, rbufs.at[0], ksend_sems.at[6],
                           relay_recv_sems.at[0], 3).start()
                slice_rdma(vstage, 6, rbufs.at[1], vsend_sems.at[6],
                           relay_recv_sems.at[1], 3).start()
                kdir(2).start()
                vdir(2).start()
                kdir(1).start()
                vdir(1).start()

                slice_rdma(kstage, 5, rbufs.at[0], ksend_sems.at[5],
                           relay_recv_sems.at[0], 4).wait_send()
                slice_rdma(vstage, 5, rbufs.at[1], vsend_sems.at[5],
                           relay_recv_sems.at[1], 4).wait_send()
                slice_rdma(vstage, 7, rbufs.at[2], vsend_sems.at[7],
                           relay_recv_sems.at[2], 4).wait_send()
                kdir(4).start()
                vdir(4).start()

                slice_rdma(kstage, 6, rbufs.at[0], ksend_sems.at[6],
                           relay_recv_sems.at[0], 3).wait_send()
                slice_rdma(vstage, 6, rbufs.at[1], vsend_sems.at[6],
                           relay_recv_sems.at[1], 3).wait_send()
                kdir(3).start()
                vdir(3).start()
                kdir(7).start()

                kbuf[...] = kstage[:, :, 0:HQ_LOC * DH]
                vbuf[...] = vstage[:, :, 0:HQ_LOC * DH]

        def ph1_relay():
            @pl.when(me == 4)
            def _():
                relay_wait(0)
                fwd(0, kbuf, krecv_sem, 5).start()
                relay_wait(1)
                fwd(1, vbuf, vrecv_sem, 5).start()
                relay_wait(2)
                fwd(2, vbuf, vrecv_sem, 7).start()

            @pl.when(me == 3)
            def _():
                relay_wait(0)
                fwd(0, kbuf, krecv_sem, 6).start()
                relay_wait(1)
                fwd(1, vbuf, vrecv_sem, 6).start()

        def ph2_qproj():
            wq = wq_ref[...].astype(BF16)
            qs = []
            for b in range(B):
                xb = x_ref[b].astype(BF16)
                q = jnp.dot(xb, wq, preferred_element_type=jnp.float32)
                qs.append((q * 0.125).astype(BF16))
            return qs

        def ph_kv_wait():
            @pl.when(me != 0)
            def _():
                pltpu.make_async_remote_copy(
                    src_ref=kbuf, dst_ref=kbuf,
                    send_sem=ksend_sems.at[0], recv_sem=krecv_sem,
                    device_id=(0,), device_id_type=MESH).wait_recv()
                pltpu.make_async_remote_copy(
                    src_ref=vbuf, dst_ref=vbuf,
                    send_sem=vsend_sems.at[0], recv_sem=vrecv_sem,
                    device_id=(0,), device_id_type=MESH).wait_recv()

        def ph3_attn(qs):
            row = lax.broadcasted_iota(jnp.int32, (SQ, SKV_LOC), 0) // BLK
            col = lax.broadcasted_iota(jnp.int32, (SQ, SKV_LOC), 1) // BLK
            mask = col <= row
            wo = wo_ref[...].astype(BF16)
            for b in range(B):
                kb2 = kbuf[b]
                vb2 = vbuf[b]
                acc = jnp.zeros((SQ, DM), jnp.float32)
                for h in range(HQ_LOC):
                    qh = qs[b][:, DH * h:DH * (h + 1)]
                    kh = kb2[:, DH * h:DH * (h + 1)]
                    s = lax.dot_general(
                        qh, kh, (((1,), (1,)), ((), ())),
                        preferred_element_type=jnp.float32)
                    s = jnp.where(mask, s, -1e9)
                    mx = jnp.max(s, axis=1, keepdims=True)
                    w = jnp.exp(s - mx)
                    w = (w / jnp.sum(w, axis=1, keepdims=True)).astype(BF16)
                    ch = jnp.dot(w, vb2[:, DH * h:DH * (h + 1)],
                                 preferred_element_type=jnp.float32)
                    acc = acc + jnp.dot(
                        ch.astype(BF16), wo[DH * h:DH * (h + 1), :],
                        preferred_element_type=jnp.float32)
                part[b] = acc.astype(BF16)

        def rs_rdma(s, d):
            return pltpu.make_async_remote_copy(
                src_ref=part.at[:, pl.ds(SC * d, SC), :],
                dst_ref=rsbuf.at[s],
                send_sem=rssend_sems.at[d], recv_sem=rsrecv_sems.at[s],
                device_id=(d,), device_id_type=MESH)

        def ag_rdma(s, d):
            return pltpu.make_async_remote_copy(
                src_ref=agstage, dst_ref=agbuf.at[s],
                send_sem=agsend_sems.at[d], recv_sem=agrecv_sems.at[s],
                device_id=(d,), device_id_type=MESH)

        def ph4_rs_issue():
            for s in range(N_DEV):
                @pl.when(me == s)
                def _(s=s):
                    rsbuf[s] = part[:, SC * s:SC * (s + 1), :]
                    for d in range(N_DEV):
                        if d != s:
                            rs_rdma(s, d).start()

        def ph4_rs_wait():
            for s in range(N_DEV):
                @pl.when(me != s)
                def _(s=s):
                    rs_rdma(s, 0).wait_recv()

        def ph4_reduce():
            red = rsbuf[0].astype(jnp.float32)
            for s in range(1, N_DEV):
                red = red + rsbuf[s].astype(jnp.float32)
            agstage[...] = red.astype(BF16)
            return red

        def ph4_ag_issue(red):
            for s in range(N_DEV):
                @pl.when(me == s)
                def _(s=s):
                    out_ref[:, SC * s:SC * (s + 1), :] = red
                    for d in range(N_DEV):
                        if d != s:
                            ag_rdma(s, d).start()
            for s in range(N_DEV):
                @pl.when(me == s)
                def _(s=s):
                    for d in range(N_DEV):
                        if d != s:
                            rs_rdma(s, d).wait_send()

        def ph4_ag_wait():
            for s in range(N_DEV):
                @pl.when(me != s)
                def _(s=s):
                    ag_rdma(s, 0).wait_recv()
                    out_ref[:, SC * s:SC * (s + 1), :] = (
                        agbuf[s].astype(jnp.float32))
            for s in range(N_DEV):
                @pl.when(me == s)
                def _(s=s):
                    for d in range(N_DEV):
                        if d != s:
                            ag_rdma(s, d).wait_send()

        def ph5_drain():
            @pl.when(me == 0)
            def _():
                for d in (1, 2, 3, 4):
                    kdir(d).wait_send()
                    vdir(d).wait_send()
                kdir(7).wait_send()

            @pl.when(me == 4)
            def _():
                fwd(0, kbuf, krecv_sem, 5).wait_send()
                fwd(1, vbuf, vrecv_sem, 5).wait_send()
                fwd(2, vbuf, vrecv_sem, 7).wait_send()

            @pl.when(me == 3)
            def _():
                fwd(0, kbuf, krecv_sem, 6).wait_send()
                fwd(1, vbuf, vrecv_sem, 6).wait_send()

        with jax.named_scope("p1_bcast_issue"):
            ph1_bcast_issue()
        with jax.named_scope("p1_relay"):
            ph1_relay()
        with jax.named_scope("p2_qproj"):
            qs = ph2_qproj()
        with jax.named_scope("p2_kv_wait"):
            ph_kv_wait()
        with jax.named_scope("p3_attn"):
            ph3_attn(qs)
        with jax.named_scope("p4_rs_issue"):
            ph4_rs_issue()
        with jax.named_scope("p4_rs_wait"):
            ph4_rs_wait()
        with jax.named_scope("p4_reduce"):
            red = ph4_reduce()
        with jax.named_scope("p4_ag_issue"):
            ph4_ag_issue(red)
        with jax.named_scope("p4_ag_wait"):
            ph4_ag_wait()
        with jax.named_scope("p5_drain"):
            ph5_drain()

    return pl.pallas_call(
        body,
        out_shape=jax.ShapeDtypeStruct((B, SQ, DM), jnp.float32),
        in_specs=[
            pl.BlockSpec(memory_space=pltpu.VMEM),
            pl.BlockSpec(memory_space=pltpu.VMEM),
            pl.BlockSpec(memory_space=pl.ANY),
            pl.BlockSpec(memory_space=pl.ANY),
            pl.BlockSpec(memory_space=pltpu.VMEM),
        ],
        out_specs=pl.BlockSpec(memory_space=pltpu.VMEM),
        scratch_shapes=[
            pltpu.VMEM((B, SKV_LOC, HQ * DH), jnp.float32),
            pltpu.VMEM((B, SKV_LOC, HQ * DH), jnp.float32),
            pltpu.VMEM((B, SKV_LOC, HQ * DH), BF16),
            pltpu.VMEM((B, SKV_LOC, HQ * DH), BF16),
            pltpu.VMEM((B, SKV_LOC, HQ_LOC * DH), BF16),
            pltpu.VMEM((B, SKV_LOC, HQ_LOC * DH), BF16),
            pltpu.VMEM((3, B, SKV_LOC, HQ_LOC * DH), BF16),
            pltpu.VMEM((B, SQ, DM), BF16),
            pltpu.VMEM((N_DEV, B, SC, DM), BF16),
            pltpu.VMEM((B, SC, DM), BF16),
            pltpu.VMEM((N_DEV, B, SC, DM), BF16),
            pltpu.SemaphoreType.DMA((2,)),
            pltpu.SemaphoreType.DMA((N_DEV,)),
            pltpu.SemaphoreType.DMA((N_DEV,)),
            pltpu.SemaphoreType.DMA,
            pltpu.SemaphoreType.DMA,
            pltpu.SemaphoreType.DMA((3,)),
            pltpu.SemaphoreType.DMA((3,)),
            pltpu.SemaphoreType.DMA((N_DEV,)),
            pltpu.SemaphoreType.DMA((N_DEV,)),
            pltpu.SemaphoreType.DMA((N_DEV,)),
            pltpu.SemaphoreType.DMA((N_DEV,)),
        ],
    )(x, Wq,
       K_ext.reshape(B, SKV_LOC, HQ * DH),
       V_ext.reshape(B, SKV_LOC, HQ * DH),
       Wo)


# device time: 42008 ns/iter; 2.5720x vs baseline; 1.1145x over previous
import jax
import jax.numpy as jnp
from jax import lax
from jax.experimental import pallas as pl
from jax.experimental.pallas import tpu as pltpu

N_DEV = 8
B = 2
SQ = 256
DM = 512
DH = 64
HQ_LOC = 4
HQ = N_DEV * HQ_LOC
SKV_LOC = 256
BLK = 64
BF16 = jnp.bfloat16
SC = SQ // N_DEV

MESH = pl.DeviceIdType.MESH


def kernel(x, Wq, K_ext, V_ext, Wo):
    def body(x_ref, wq_ref, k_ref, v_ref, wo_ref, out_ref,
             kf32, vf32, kstage, vstage, kbuf, vbuf, rbufs,
             part, rsbuf, agstage, agbuf,
             copy_sems, ksend_sems, vsend_sems, krecv_sem, vrecv_sem,
             relay_recv_sems, fwd_send_sems,
             rssend_sems, rsrecv_sems, agsend_sems, agrecv_sems):
        me = lax.axis_index("i")

        def slice_rdma(stage, d, dst, send_sem, recv_sem, target):
            return pltpu.make_async_remote_copy(
                src_ref=stage.at[:, :, pl.ds(HQ_LOC * DH * d, HQ_LOC * DH)],
                dst_ref=dst, send_sem=send_sem, recv_sem=recv_sem,
                device_id=(target,), device_id_type=MESH)

        def kdir(d):
            return slice_rdma(kstage, d, kbuf, ksend_sems.at[d], krecv_sem, d)

        def vdir(d):
            return slice_rdma(vstage, d, vbuf, vsend_sems.at[d], vrecv_sem, d)

        def fwd(slot, dst, recv_sem, target):
            return pltpu.make_async_remote_copy(
                src_ref=rbufs.at[slot], dst_ref=dst,
                send_sem=fwd_send_sems.at[slot], recv_sem=recv_sem,
                device_id=(target,), device_id_type=MESH)

        def relay_wait(slot):
            pltpu.make_async_remote_copy(
                src_ref=rbufs.at[slot], dst_ref=rbufs.at[slot],
                send_sem=fwd_send_sems.at[slot],
                recv_sem=relay_recv_sems.at[slot],
                device_id=(0,), device_id_type=MESH).wait_recv()

        def ph1_bcast_issue():
            @pl.when(me == 0)
            def _():
                ck = pltpu.make_async_copy(k_ref, kf32, copy_sems.at[0])
                cv = pltpu.make_async_copy(v_ref, vf32, copy_sems.at[1])
                ck.start()
                cv.start()
                ck.wait()
                cv.wait()
                kstage[...] = kf32[...].astype(BF16)
                vstage[...] = vf32[...].astype(BF16)

                slice_rdma(kstage, 5, rbufs.at[0], ksend_sems.at[5],
                           relay_recv_sems.at[0], 4).start()
                slice_rdma(vstage, 5, rbufs.at[1], vsend_sems.at[5],
                           relay_recv_sems.at[1], 4).start()
                slice_rdma(vstage, 7, rbufs.at[2], vsend_sems.at[7],
                           relay_recv_sems.at[2], 4).start()
                slice_rdma(kstage, 6, rbufs.at[0], ksend_sems.at[6],
                           relay_recv_sems.at[0], 3).start()
                slice_rdma(vstage, 6, rbufs.at[1], vsend_sems.at[6],
                           relay_recv_sems.at[1], 3).start()
                kdir(7).start()
                kdir(2).start()
                kdir(1).start()

                kbuf[...] = kstage[:, :, 0:HQ_LOC * DH]
                vbuf[...] = vstage[:, :, 0:HQ_LOC * DH]

                kdir(2).wait_send()
                kdir(1).wait_send()
                vdir(2).start()
                vdir(1).start()

                slice_rdma(kstage, 5, rbufs.at[0], ksend_sems.at[5],
                           relay_recv_sems.at[0], 4).wait_send()
                slice_rdma(vstage, 5, rbufs.at[1], vsend_sems.at[5],
                           relay_recv_sems.at[1], 4).wait_send()
                slice_rdma(vstage, 7, rbufs.at[2], vsend_sems.at[7],
                           relay_recv_sems.at[2], 4).wait_send()
                kdir(4).start()
                vdir(4).start()

                slice_rdma(kstage, 6, rbufs.at[0], ksend_sems.at[6],
                           relay_recv_sems.at[0], 3).wait_send()
                slice_rdma(vstage, 6, rbufs.at[1], vsend_sems.at[6],
                           relay_recv_sems.at[1], 3).wait_send()
                kdir(7).wait_send()
                kdir(3).start()
                vdir(3).start()

        def ph1_relay():
            @pl.when(me == 4)
            def _():
                relay_wait(0)
                fwd(0, kbuf, krecv_sem, 5).start()
                relay_wait(1)
                fwd(1, vbuf, vrecv_sem, 5).start()
                relay_wait(2)
                fwd(2, vbuf, vrecv_sem, 7).start()

            @pl.when(me == 3)
            def _():
                relay_wait(0)
                fwd(0, kbuf, krecv_sem, 6).start()
                relay_wait(1)
                fwd(1, vbuf, vrecv_sem, 6).start()

        def ph2_qproj():
            wq = wq_ref[...].astype(BF16)
            qs = []
            for b in range(B):
                xb = x_ref[b].astype(BF16)
                q = jnp.dot(xb, wq, preferred_element_type=jnp.float32)
                qs.append((q * 0.125).astype(BF16))
            return qs

        def ph_k_wait():
            @pl.when(me != 0)
            def _():
                pltpu.make_async_remote_copy(
                    src_ref=kbuf, dst_ref=kbuf,
                    send_sem=ksend_sems.at[0], recv_sem=krecv_sem,
                    device_id=(0,), device_id_type=MESH).wait_recv()

        def ph_v_wait():
            @pl.when(me != 0)
            def _():
                pltpu.make_async_remote_copy(
                    src_ref=vbuf, dst_ref=vbuf,
                    send_sem=vsend_sems.at[0], recv_sem=vrecv_sem,
                    device_id=(0,), device_id_type=MESH).wait_recv()

        def ph3a_scores(qs):
            row = lax.broadcasted_iota(jnp.int32, (SQ, SKV_LOC), 0) // BLK
            col = lax.broadcasted_iota(jnp.int32, (SQ, SKV_LOC), 1) // BLK
            mask = col <= row
            ws = []
            for b in range(B):
                kb2 = kbuf[b]
                wb = []
                for h in range(HQ_LOC):
                    qh = qs[b][:, DH * h:DH * (h + 1)]
                    kh = kb2[:, DH * h:DH * (h + 1)]
                    s = lax.dot_general(
                        qh, kh, (((1,), (1,)), ((), ())),
                        preferred_element_type=jnp.float32)
                    s = jnp.where(mask, s, -1e9)
                    mx = jnp.max(s, axis=1, keepdims=True)
                    w = jnp.exp(s - mx)
                    wb.append(
                        (w / jnp.sum(w, axis=1, keepdims=True)).astype(BF16))
                ws.append(wb)
            return ws

        def ph3b_ctx(ws):
            wo = wo_ref[...].astype(BF16)
            for b in range(B):
                vb2 = vbuf[b]
                acc = jnp.zeros((SQ, DM), jnp.float32)
                for h in range(HQ_LOC):
                    ch = jnp.dot(ws[b][h], vb2[:, DH * h:DH * (h + 1)],
                                 preferred_element_type=jnp.float32)
                    acc = acc + jnp.dot(
                        ch.astype(BF16), wo[DH * h:DH * (h + 1), :],
                        preferred_element_type=jnp.float32)
                part[b] = acc.astype(BF16)

        def rs_rdma(s, d):
            return pltpu.make_async_remote_copy(
                src_ref=part.at[:, pl.ds(SC * d, SC), :],
                dst_ref=rsbuf.at[s],
                send_sem=rssend_sems.at[d], recv_sem=rsrecv_sems.at[s],
                device_id=(d,), device_id_type=MESH)

        def ag_rdma(s, d):
            return pltpu.make_async_remote_copy(
                src_ref=agstage, dst_ref=agbuf.at[s],
                send_sem=agsend_sems.at[d], recv_sem=agrecv_sems.at[s],
                device_id=(d,), device_id_type=MESH)

        def ph4_rs_issue():
            for s in range(N_DEV):
                @pl.when(me == s)
                def _(s=s):
                    rsbuf[s] = part[:, SC * s:SC * (s + 1), :]
                    for d in range(N_DEV):
                        if d != s:
                            rs_rdma(s, d).start()

        def ph4_rs_wait():
            for s in range(N_DEV):
                @pl.when(me != s)
                def _(s=s):
                    rs_rdma(s, 0).wait_recv()

        def ph4_reduce():
            red = rsbuf[0].astype(jnp.float32)
            for s in range(1, N_DEV):
                red = red + rsbuf[s].astype(jnp.float32)
            agstage[...] = red.astype(BF16)
            return red

        def ph4_ag_issue(red):
            for s in range(N_DEV):
                @pl.when(me == s)
                def _(s=s):
                    out_ref[:, SC * s:SC * (s + 1), :] = red
                    for d in range(N_DEV):
                        if d != s:
                            ag_rdma(s, d).start()
            for s in range(N_DEV):
                @pl.when(me == s)
                def _(s=s):
                    for d in range(N_DEV):
                        if d != s:
                            rs_rdma(s, d).wait_send()

        def ph4_ag_wait():
            for s in range(N_DEV):
                @pl.when(me != s)
                def _(s=s):
                    ag_rdma(s, 0).wait_recv()
                    out_ref[:, SC * s:SC * (s + 1), :] = (
                        agbuf[s].astype(jnp.float32))
            for s in range(N_DEV):
                @pl.when(me == s)
                def _(s=s):
                    for d in range(N_DEV):
                        if d != s:
                            ag_rdma(s, d).wait_send()

        def ph5_drain():
            @pl.when(me == 0)
            def _():
                for d in (1, 2, 3, 4):
                    vdir(d).wait_send()
                kdir(3).wait_send()
                kdir(4).wait_send()

            @pl.when(me == 4)
            def _():
                fwd(0, kbuf, krecv_sem, 5).wait_send()
                fwd(1, vbuf, vrecv_sem, 5).wait_send()
                fwd(2, vbuf, vrecv_sem, 7).wait_send()

            @pl.when(me == 3)
            def _():
                fwd(0, kbuf, krecv_sem, 6).wait_send()
                fwd(1, vbuf, vrecv_sem, 6).wait_send()

        with jax.named_scope("p1_bcast_issue"):
            ph1_bcast_issue()
        with jax.named_scope("p1_relay"):
            ph1_relay()
        with jax.named_scope("p2_qproj"):
            qs = ph2_qproj()
        with jax.named_scope("p2_k_wait"):
            ph_k_wait()
        with jax.named_scope("p3_scores"):
            ws = ph3a_scores(qs)
        with jax.named_scope("p2_v_wait"):
            ph_v_wait()
        with jax.named_scope("p3_ctx"):
            ph3b_ctx(ws)
        with jax.named_scope("p4_rs_issue"):
            ph4_rs_issue()
        with jax.named_scope("p4_rs_wait"):
            ph4_rs_wait()
        with jax.named_scope("p4_reduce"):
            red = ph4_reduce()
        with jax.named_scope("p4_ag_issue"):
            ph4_ag_issue(red)
        with jax.named_scope("p4_ag_wait"):
            ph4_ag_wait()
        with jax.named_scope("p5_drain"):
            ph5_drain()

    return pl.pallas_call(
        body,
        out_shape=jax.ShapeDtypeStruct((B, SQ, DM), jnp.float32),
        in_specs=[
            pl.BlockSpec(memory_space=pltpu.VMEM),
            pl.BlockSpec(memory_space=pltpu.VMEM),
            pl.BlockSpec(memory_space=pl.ANY),
            pl.BlockSpec(memory_space=pl.ANY),
            pl.BlockSpec(memory_space=pltpu.VMEM),
        ],
        out_specs=pl.BlockSpec(memory_space=pltpu.VMEM),
        scratch_shapes=[
            pltpu.VMEM((B, SKV_LOC, HQ * DH), jnp.float32),
            pltpu.VMEM((B, SKV_LOC, HQ * DH), jnp.float32),
            pltpu.VMEM((B, SKV_LOC, HQ * DH), BF16),
            pltpu.VMEM((B, SKV_LOC, HQ * DH), BF16),
            pltpu.VMEM((B, SKV_LOC, HQ_LOC * DH), BF16),
            pltpu.VMEM((B, SKV_LOC, HQ_LOC * DH), BF16),
            pltpu.VMEM((3, B, SKV_LOC, HQ_LOC * DH), BF16),
            pltpu.VMEM((B, SQ, DM), BF16),
            pltpu.VMEM((N_DEV, B, SC, DM), BF16),
            pltpu.VMEM((B, SC, DM), BF16),
            pltpu.VMEM((N_DEV, B, SC, DM), BF16),
            pltpu.SemaphoreType.DMA((2,)),
            pltpu.SemaphoreType.DMA((N_DEV,)),
            pltpu.SemaphoreType.DMA((N_DEV,)),
            pltpu.SemaphoreType.DMA,
            pltpu.SemaphoreType.DMA,
            pltpu.SemaphoreType.DMA((3,)),
            pltpu.SemaphoreType.DMA((3,)),
            pltpu.SemaphoreType.DMA((N_DEV,)),
            pltpu.SemaphoreType.DMA((N_DEV,)),
            pltpu.SemaphoreType.DMA((N_DEV,)),
            pltpu.SemaphoreType.DMA((N_DEV,)),
        ],
    )(x, Wq,
       K_ext.reshape(B, SKV_LOC, HQ * DH),
       V_ext.reshape(B, SKV_LOC, HQ * DH),
       Wo)
